# baseline (speedup 1.0000x reference)
import functools

import jax
import jax.numpy as jnp
from jax.experimental import pallas as pl

_STRIPE_W = 128


def _stripe_kernel(x_ref, o_ref, *, inv_n):
    x = x_ref[...]
    mean = jnp.sum(x, axis=0, keepdims=True) * inv_n
    lane = jax.lax.broadcasted_iota(jnp.int32, x.shape, dimension=1)
    o_ref[...] = jnp.where(lane % 2 == 0, jnp.broadcast_to(mean, x.shape), x)


def kernel(inputs):
    orig_shape = inputs.shape
    c = orig_shape[-1]
    rows = 1
    for d in orig_shape[:-1]:
        rows *= d
    x = inputs.reshape(rows, c)
    nstripes = c // _STRIPE_W

    out = pl.pallas_call(
        functools.partial(_stripe_kernel, inv_n=1.0 / rows),
        grid=(nstripes,),
        in_specs=[pl.BlockSpec((rows, _STRIPE_W), lambda s: (0, s))],
        out_specs=pl.BlockSpec((rows, _STRIPE_W), lambda s: (0, s)),
        out_shape=jax.ShapeDtypeStruct((rows, c), jnp.float32),
    )(x)

    return out.reshape(orig_shape)


# speedup vs baseline: 1.0708x; 1.0708x over previous
import functools

import jax
import jax.numpy as jnp
from jax.experimental import pallas as pl

_STRIPE_W = 256


def _stripe_kernel(x_ref, o_ref, *, inv_n):
    x = x_ref[...]
    mean = jnp.sum(x, axis=0, keepdims=True) * inv_n
    lane = jax.lax.broadcasted_iota(jnp.int32, x.shape, dimension=1)
    o_ref[...] = jnp.where(lane % 2 == 0, jnp.broadcast_to(mean, x.shape), x)


def kernel(inputs):
    orig_shape = inputs.shape
    c = orig_shape[-1]
    rows = 1
    for d in orig_shape[:-1]:
        rows *= d
    x = inputs.reshape(rows, c)
    nstripes = c // _STRIPE_W

    out = pl.pallas_call(
        functools.partial(_stripe_kernel, inv_n=1.0 / rows),
        grid=(nstripes,),
        in_specs=[pl.BlockSpec((rows, _STRIPE_W), lambda s: (0, s))],
        out_specs=pl.BlockSpec((rows, _STRIPE_W), lambda s: (0, s)),
        out_shape=jax.ShapeDtypeStruct((rows, c), jnp.float32),
    )(x)

    return out.reshape(orig_shape)
